# baseline (speedup 1.0000x reference)
import functools

import jax
import jax.numpy as jnp
import numpy as np
from jax import lax
from jax.experimental import pallas as pl
from jax.experimental.pallas import tpu as pltpu
from jax.experimental.pallas import tpu_sc as plsc

N = 10000
N_PAD = 10240
E = 320000
F = 128
FH = F // 2

NC = 2
NS = 16
NW = NC * NS

C = 80
CHUNKS_PER_TILE = E // (NS * C)
XROWS = N // NS
XCONV = 125

MM_BLOCK = 2000
CB_BLOCK = 2000

_PERM = np.empty((F,), dtype=np.int32)
for _p in range(F):
    _blk = (_p // 32) * 32
    _i = _p % 32
    _PERM[_p] = _blk + (_i // 2) + (16 if _i % 2 else 0)


def _matmul_body(x_ref, w_ref, o_ref):
    xw = jnp.dot(x_ref[...], w_ref[...], preferred_element_type=jnp.float32)
    o_ref[:, :F] = xw.astype(jnp.bfloat16)
    o_ref[:, F:] = jnp.zeros((MM_BLOCK, F), jnp.bfloat16)


def _matmul(X, W):
    return pl.pallas_call(
        _matmul_body,
        grid=(N // MM_BLOCK,),
        in_specs=[
            pl.BlockSpec((MM_BLOCK, F), lambda i: (i, 0)),
            pl.BlockSpec((F, F), lambda i: (0, 0)),
        ],
        out_specs=pl.BlockSpec((MM_BLOCK, 2 * F), lambda i: (i, 0)),
        out_shape=jax.ShapeDtypeStruct((N, 2 * F), jnp.bfloat16),
    )(X, W)


def _sc_body(xw_hbm, packed_hbm, vals_hbm, b_hbm, out_hbm,
             packed_v, vals_v, sr0, sr1, sr2, sr3, dr0, dr1, dr2, dr3,
             g0, g1, s0, s1, zf32, bias_v, acc,
             gsem0, gsem1, ssem0, ssem1):
    cid = lax.axis_index("c")
    sid = lax.axis_index("s")
    pltpu.sync_copy(b_hbm.at[pl.ds(cid * FH, FH)], bias_v)

    pltpu.sync_copy(packed_hbm.at[sid], packed_v)
    pltpu.sync_copy(vals_hbm.at[sid], vals_v)

    zero = jnp.zeros((16,), jnp.float32)

    def zero_row(i, carry):
        for j in range(FH // 16):
            zf32[i, pl.ds(j * 16, 16)] = zero
        return carry

    lax.fori_loop(0, XCONV, zero_row, 0)
    for k in range(XROWS // XCONV):
        pltpu.sync_copy(zf32, acc.at[pl.ds(sid * XROWS + k * XCONV, XCONV)])

    plsc.subcore_barrier()

    srings = (sr0, sr1, sr2, sr3)
    drings = (dr0, dr1, dr2, dr3)
    gbufs = (g0, g1)
    sbufs = (s0, s1)
    gsems = (gsem0, gsem1)
    ssems = (ssem0, ssem1)

    def unpack_idx(c, r):
        for g in range(C // 16):
            sl = pl.ds(g * 16, 16)
            p = packed_v[c, sl]
            srings[r][sl] = ((p & 0x3FFF) << 2) + cid
            drings[r][sl] = p >> 14

    def start_gather(r, b):
        pltpu.async_copy(xw_hbm.at[srings[r]], gbufs[b], gsems[b])

    def wait_gather(r, b):
        pltpu.make_async_copy(xw_hbm.at[srings[r]], gbufs[b],
                              gsems[b]).wait()

    def start_scatter(r, b):
        pltpu.async_copy(sbufs[b], acc.at[drings[r]], ssems[b], add=True)

    def wait_scatter(r, b):
        pltpu.make_async_copy(sbufs[b], acc.at[drings[r]], ssems[b]).wait()

    def scale(c, b):
        gbuf, sbuf = gbufs[b], sbufs[b]
        for g in range(C // 16):
            vv = vals_v[c, pl.ds(g * 16, 16)]
            for l in range(16):
                v = vv[l]
                base = g * 16 + l
                for j in range(FH // 32):
                    xi = plsc.bitcast(gbuf[base, pl.ds(j * 32, 32)],
                                      jnp.int32)
                    a = plsc.bitcast(xi << 16, jnp.float32)
                    bb = plsc.bitcast(xi & jnp.int32(-65536), jnp.float32)
                    sbuf[base, pl.ds(j * 32, 16)] = a * v
                    sbuf[base, pl.ds(j * 32 + 16, 16)] = bb * v

    CH = CHUNKS_PER_TILE

    unpack_idx(0, 0)
    start_gather(0, 0)
    unpack_idx(1, 1)
    start_gather(1, 1)
    wait_gather(0, 0)
    scale(0, 0)
    start_scatter(0, 0)
    unpack_idx(2, 2)
    start_gather(2, 0)
    wait_gather(1, 1)
    scale(1, 1)
    start_scatter(1, 1)

    @pl.loop(2, CH - 4, step=4)
    def _quads(c):
        for k in range(4):
            cc = c + k
            rk = (2 + k) % 4
            rn = (3 + k) % 4
            bk = k % 2
            bn = (k + 1) % 2
            unpack_idx(cc + 1, rn)
            start_gather(rn, bn)
            wait_gather(rk, bk)
            wait_scatter(rk, bk)
            scale(cc, bk)
            start_scatter(rk, bk)

    for cc in range(CH - 4, CH):
        rk = cc % 4
        rn = (cc + 1) % 4
        bk = cc % 2
        bn = (cc + 1) % 2
        if cc + 1 < CH:
            unpack_idx(cc + 1, rn)
            start_gather(rn, bn)
        wait_gather(rk, bk)
        wait_scatter(rk, bk)
        scale(cc, bk)
        start_scatter(rk, bk)
    wait_scatter((CH - 2) % 4, (CH - 2) % 2)
    wait_scatter((CH - 1) % 4, (CH - 1) % 2)
    plsc.subcore_barrier()

    bvec = [bias_v[pl.ds(j * 16, 16)] for j in range(FH // 16)]
    for k in range(XROWS // XCONV):
        r0 = sid * XROWS + k * XCONV
        pltpu.sync_copy(acc.at[pl.ds(r0, XCONV)], zf32)

        def pub_row(r, carry):
            for j in range(FH // 16):
                sl = pl.ds(j * 16, 16)
                zf32[r, sl] = zf32[r, sl] + bvec[j]
            return carry

        lax.fori_loop(0, XCONV, pub_row, 0)
        pltpu.sync_copy(zf32, out_hbm.at[pl.ds(r0, XCONV),
                                         pl.ds(cid * FH, FH)])


_sc_scatter = functools.partial(
    pl.kernel,
    out_type=jax.ShapeDtypeStruct((N, F), jnp.float32),
    mesh=plsc.VectorSubcoreMesh(core_axis_name="c", subcore_axis_name="s"),
    compiler_params=pltpu.CompilerParams(use_tc_tiling_on_sc=False,
                                         needs_layout_passes=False),
    scratch_types=[
        pltpu.VMEM((CHUNKS_PER_TILE, C), jnp.int32),
        pltpu.VMEM((CHUNKS_PER_TILE, C), jnp.float32),
        pltpu.VMEM((C,), jnp.int32),
        pltpu.VMEM((C,), jnp.int32),
        pltpu.VMEM((C,), jnp.int32),
        pltpu.VMEM((C,), jnp.int32),
        pltpu.VMEM((C,), jnp.int32),
        pltpu.VMEM((C,), jnp.int32),
        pltpu.VMEM((C,), jnp.int32),
        pltpu.VMEM((C,), jnp.int32),
        pltpu.VMEM((C, FH), jnp.bfloat16),
        pltpu.VMEM((C, FH), jnp.bfloat16),
        pltpu.VMEM((C, FH), jnp.float32),
        pltpu.VMEM((C, FH), jnp.float32),
        pltpu.VMEM((XCONV, FH), jnp.float32),
        pltpu.VMEM((FH,), jnp.float32),
        pltpu.VMEM_SHARED((N, FH), jnp.float32),
        pltpu.SemaphoreType.DMA,
        pltpu.SemaphoreType.DMA,
        pltpu.SemaphoreType.DMA,
        pltpu.SemaphoreType.DMA,
    ],
)(_sc_body)


def kernel(X, edge_index, A_values, W, b):
    XW = _matmul(X, W[:, _PERM]).reshape(4 * N, FH)
    shape3 = (NS, CHUNKS_PER_TILE, C)
    ei = edge_index.astype(jnp.int32)
    packed = ((ei[0] << 14) | ei[1]).reshape(shape3)
    return _sc_scatter(XW, packed, A_values.reshape(shape3), b)


# speedup vs baseline: 1.2033x; 1.2033x over previous
import functools

import jax
import jax.numpy as jnp
from jax import lax
from jax.experimental import pallas as pl
from jax.experimental.pallas import tpu as pltpu
from jax.experimental.pallas import tpu_sc as plsc

N = 10000
N_PAD = 10240
E = 320000
F = 128
FH = F // 2

NC = 2
NS = 16
NW = NC * NS

C = 80
CHUNKS_PER_TILE = E // (NS * C)
ROWS_PER_TILE = N_PAD // NS
ZROWS = 128

MM_BLOCK = 2000
CB_BLOCK = 2000


def _matmul_body(x_ref, w_ref, o_ref):
    o_ref[...] = jnp.dot(x_ref[...], w_ref[...],
                         preferred_element_type=jnp.float32)


def _matmul(X, W):
    return pl.pallas_call(
        _matmul_body,
        grid=(N // MM_BLOCK,),
        in_specs=[
            pl.BlockSpec((MM_BLOCK, F), lambda i: (i, 0)),
            pl.BlockSpec((F, F), lambda i: (0, 0)),
        ],
        out_specs=pl.BlockSpec((MM_BLOCK, F), lambda i: (i, 0)),
        out_shape=jax.ShapeDtypeStruct((N, F), jnp.float32),
    )(X, W)


def _combine_body(p_ref, b_ref, o_ref):
    o_ref[:, :FH] = p_ref[0] + b_ref[:, :FH]
    o_ref[:, FH:] = p_ref[1] + b_ref[:, FH:]


def _combine(partials, b2d):
    return pl.pallas_call(
        _combine_body,
        grid=(N // CB_BLOCK,),
        in_specs=[
            pl.BlockSpec((NC, CB_BLOCK, FH), lambda i: (0, i, 0)),
            pl.BlockSpec((1, F), lambda i: (0, 0)),
        ],
        out_specs=pl.BlockSpec((CB_BLOCK, F), lambda i: (i, 0)),
        out_shape=jax.ShapeDtypeStruct((N, F), jnp.float32),
    )(partials, b2d)


def _sc_body(xw_hbm, ei_hbm, vals_hbm, out_hbm,
             src_v, dst_v, vals_v, g0, g1, s0, s1, zbuf, acc,
             gsem0, gsem1, ssem0, ssem1):
    cid = lax.axis_index("c")
    sid = lax.axis_index("s")

    pltpu.sync_copy(ei_hbm.at[0, sid], dst_v)
    pltpu.sync_copy(ei_hbm.at[1, sid], src_v)
    pltpu.sync_copy(vals_hbm.at[sid], vals_v)

    def xform_row(c, carry):
        for g in range(C // 16):
            sl = pl.ds(g * 16, 16)
            src_v[c, sl] = src_v[c, sl] * 2 + cid
        return carry

    lax.fori_loop(0, CHUNKS_PER_TILE, xform_row, 0)

    zero = jnp.zeros((16,), jnp.float32)

    def zero_row(i, carry):
        for j in range(FH // 16):
            zbuf[i, pl.ds(j * 16, 16)] = zero
        return carry

    lax.fori_loop(0, ZROWS, zero_row, 0)
    for k in range(ROWS_PER_TILE // ZROWS):
        pltpu.sync_copy(zbuf, acc.at[pl.ds(sid * ROWS_PER_TILE + k * ZROWS,
                                           ZROWS)])
    plsc.subcore_barrier()

    def start_gather(c, buf, sem):
        pltpu.async_copy(xw_hbm.at[src_v.at[c]], buf, sem)

    def wait_gather(c, buf, sem):
        pltpu.make_async_copy(xw_hbm.at[src_v.at[c]], buf, sem).wait()

    def start_scatter(c, buf, sem):
        pltpu.async_copy(buf, acc.at[dst_v.at[c]], sem, add=True)

    def wait_scatter(c, buf, sem):
        pltpu.make_async_copy(buf, acc.at[dst_v.at[c]], sem).wait()

    def scale(c, gbuf, sbuf):
        for g in range(C // 16):
            vv = vals_v[c, pl.ds(g * 16, 16)]
            for l in range(16):
                v = vv[l]
                base = g * 16 + l
                for j in range(FH // 16):
                    sl = pl.ds(j * 16, 16)
                    sbuf[base, sl] = gbuf[base, sl] * v

    CH = CHUNKS_PER_TILE
    start_gather(0, g0, gsem0)
    start_gather(1, g1, gsem1)
    wait_gather(0, g0, gsem0)
    scale(0, g0, s0)
    start_scatter(0, s0, ssem0)
    start_gather(2, g0, gsem0)
    wait_gather(1, g1, gsem1)
    scale(1, g1, s1)
    wait_scatter(0, s0, ssem0)
    start_scatter(1, s1, ssem1)

    @pl.loop(2, CH - 2, step=2)
    def _pairs(c):
        start_gather(c + 1, g1, gsem1)
        wait_gather(c, g0, gsem0)
        scale(c, g0, s0)
        wait_scatter(c - 1, s1, ssem1)
        start_scatter(c, s0, ssem0)
        start_gather(c + 2, g0, gsem0)
        wait_gather(c + 1, g1, gsem1)
        scale(c + 1, g1, s1)
        wait_scatter(c, s0, ssem0)
        start_scatter(c + 1, s1, ssem1)

    start_gather(CH - 1, g1, gsem1)
    wait_gather(CH - 2, g0, gsem0)
    scale(CH - 2, g0, s0)
    wait_scatter(CH - 3, s1, ssem1)
    start_scatter(CH - 2, s0, ssem0)
    wait_gather(CH - 1, g1, gsem1)
    scale(CH - 1, g1, s1)
    wait_scatter(CH - 2, s0, ssem0)
    start_scatter(CH - 1, s1, ssem1)
    wait_scatter(CH - 1, s1, ssem1)
    plsc.subcore_barrier()

    for k in range(ROWS_PER_TILE // ZROWS):
        r0 = sid * ROWS_PER_TILE + k * ZROWS
        pltpu.sync_copy(acc.at[pl.ds(r0, ZROWS)],
                        out_hbm.at[cid, pl.ds(r0, ZROWS)])


_sc_scatter = functools.partial(
    pl.kernel,
    out_type=jax.ShapeDtypeStruct((NC, N_PAD, FH), jnp.float32),
    mesh=plsc.VectorSubcoreMesh(core_axis_name="c", subcore_axis_name="s"),
    compiler_params=pltpu.CompilerParams(use_tc_tiling_on_sc=False),
    scratch_types=[
        pltpu.VMEM((CHUNKS_PER_TILE, C), jnp.int32),
        pltpu.VMEM((CHUNKS_PER_TILE, C), jnp.int32),
        pltpu.VMEM((CHUNKS_PER_TILE, C), jnp.float32),
        pltpu.VMEM((C, FH), jnp.float32),
        pltpu.VMEM((C, FH), jnp.float32),
        pltpu.VMEM((C, FH), jnp.float32),
        pltpu.VMEM((C, FH), jnp.float32),
        pltpu.VMEM((ZROWS, FH), jnp.float32),
        pltpu.VMEM_SHARED((N_PAD, FH), jnp.float32),
        pltpu.SemaphoreType.DMA,
        pltpu.SemaphoreType.DMA,
        pltpu.SemaphoreType.DMA,
        pltpu.SemaphoreType.DMA,
    ],
)(_sc_body)


def kernel(X, edge_index, A_values, W, b):
    XW = _matmul(X, W)
    xw2 = XW.reshape(2 * N, FH)
    ei4 = edge_index.astype(jnp.int32).reshape(2, NS, CHUNKS_PER_TILE, C)
    vals3 = A_values.reshape(NS, CHUNKS_PER_TILE, C)
    partials = _sc_scatter(xw2, ei4, vals3)
    return _combine(partials, b.reshape(1, F))
